# baseline (speedup 1.0000x reference)
import jax
import jax.numpy as jnp
from jax.experimental import pallas as pl
from jax.experimental.pallas import tpu as pltpu


def _moe_body(x_ref, w1g_ref, w1u_ref, w2a_ref, w2b_ref, tw_ref, ids_ref,
              out_ref):
    e = pl.program_id(0)
    x = x_ref[...]
    dn = (((1,), (1,)), ((), ()))
    kh = w2a_ref.shape[2]
    gate = jax.lax.dot_general(x, w1g_ref[0, 0], dn,
                               preferred_element_type=jnp.float32)
    up = jax.lax.dot_general(x, w1u_ref[0, 0], dn,
                             preferred_element_type=jnp.float32)
    act = gate * jax.nn.sigmoid(gate) * up
    ya = jax.lax.dot_general(act, w2a_ref[0, 0], dn,
                             preferred_element_type=jnp.float32)
    yb = jax.lax.dot_general(act, w2b_ref[0, 0], dn,
                             preferred_element_type=jnp.float32)
    sel = (ids_ref[...] == e).astype(jnp.float32)
    wpe = jnp.sum(tw_ref[...] * sel, axis=1, keepdims=True)

    @pl.when(e == 0)
    def _init():
        out_ref[:, :kh] = wpe * ya
        out_ref[:, kh:] = wpe * yb

    @pl.when(e > 0)
    def _acc():
        out_ref[:, :kh] += wpe * ya
        out_ref[:, kh:] += wpe * yb


def kernel(hidden_states, w1, w2, topk_weights, topk_ids):
    m, k = hidden_states.shape
    e_total, two_n, _ = w1.shape
    n = w2.shape[2]
    topk = topk_ids.shape[1]
    kh = k // 2
    w1r = w1.reshape(e_total, 2, n, k)
    w2r = w2.reshape(e_total, 2, kh, n)
    return pl.pallas_call(
        _moe_body,
        grid=(e_total,),
        in_specs=[
            pl.BlockSpec((m, k), lambda e: (0, 0)),
            pl.BlockSpec((1, 1, n, k), lambda e: (e, 0, 0, 0)),
            pl.BlockSpec((1, 1, n, k), lambda e: (e, 1, 0, 0)),
            pl.BlockSpec((1, 1, kh, n), lambda e: (e, 0, 0, 0)),
            pl.BlockSpec((1, 1, kh, n), lambda e: (e, 1, 0, 0)),
            pl.BlockSpec((m, topk), lambda e: (0, 0)),
            pl.BlockSpec((m, topk), lambda e: (0, 0)),
        ],
        out_specs=pl.BlockSpec((m, k), lambda e: (0, 0)),
        out_shape=jax.ShapeDtypeStruct((m, k), jnp.float32),
        compiler_params=pltpu.CompilerParams(
            dimension_semantics=("arbitrary",)),
    )(hidden_states, w1r, w1r, w2r, w2r, topk_weights, topk_ids)
